# baseline (speedup 1.0000x reference)
import functools
import math

import jax
import jax.numpy as jnp
from jax import lax
from jax.experimental import pallas as pl
from jax.experimental.pallas import tpu as pltpu
from jax.experimental.pallas import tpu_sc as plsc

_VOCAB = 100000
_D = 64
_B = 4096
_S = 200
_SCALE = math.sqrt(float(_D))

_NC = 2
_NS = 16
_NW = _NC * _NS
_W = 256
_NH = _B // _W
_NM = _NW // _NH
_NT = _S // _NM
_DI = _D // 8
_JB = _B // 128


def _sc_body(tok_hbm, table_hbm, pe_hbm, out_hbm, idx_v, pe_v, bidx_v,
             rot_v, div_v, rm_v, rows_v, obuf_v,
             semi, semg0, semg1, sems0, sems1):
    cid = lax.axis_index("c")
    sid = lax.axis_index("s")
    wid = sid * _NC + cid
    h = wid // _NM
    m = wid % _NM

    pltpu.sync_copy(pe_hbm, pe_v)
    iota = lax.iota(jnp.int32, 16)
    for c0 in range(16):
        bidx_v[c0, :] = iota + (c0 * 16)
    for t in range(16):
        rot = lax.rem(iota + t, jnp.full((16,), 16, jnp.int32))
        rot_v[t, :] = rot
        div_v[t, :] = lax.shift_right_logical(rot, jnp.full((16,), 3, jnp.int32))
        rm_v[t, :] = lax.rem(rot, jnp.full((16,), 8, jnp.int32))

    def s_of(k):
        return k * _NM + m

    def start_idx(k, q):
        pltpu.async_copy(tok_hbm.at[s_of(k), pl.ds(h * _W, _W)],
                         idx_v.at[q], semi)

    def wait_idx(k, q):
        pltpu.make_async_copy(tok_hbm.at[s_of(k), pl.ds(h * _W, _W)],
                              idx_v.at[q], semi).wait()

    def gsem(p):
        return (semg0, semg1)[p]

    def start_gather(k, p):
        for half in range(2):
            pltpu.async_copy(
                table_hbm.at[idx_v.at[p, pl.ds(half * 128, 128)]],
                rows_v.at[p, pl.ds(half * 128, 128)], gsem(p))

    def wait_gather(k, p):
        for half in range(2):
            pltpu.make_async_copy(
                table_hbm.at[idx_v.at[p, pl.ds(half * 128, 128)]],
                rows_v.at[p, pl.ds(half * 128, 128)], gsem(p)).wait()

    def ssem(p):
        return (sems0, sems1)[p]

    def start_store(k, p):
        pltpu.async_copy(obuf_v.at[p],
                         out_hbm.at[s_of(k), :, pl.ds(2 * h, 2)], ssem(p))

    def wait_store(k, p):
        pltpu.make_async_copy(obuf_v.at[p],
                              out_hbm.at[s_of(k), :, pl.ds(2 * h, 2)],
                              ssem(p)).wait()

    def compute(k, rows_p, obuf_p):
        rp = rows_v.at[rows_p]
        ob = obuf_v.at[obuf_p]
        s_idx = lax.broadcast(s_of(k), (16,))
        pats = tuple(bidx_v[c0, :] for c0 in range(16))

        @plsc.parallel_loop(0, 64, 1, unroll=4)
        def body_q(q):
            t = lax.rem(q, 16)
            d0i = lax.div(q, 16)
            rot = rot_v[t, :]
            dpat = rot + d0i * 16
            peb = plsc.load_gather(pe_v, [s_idx, dpat])
            iidx = div_v[t, :] + d0i * 2
            ridx = rm_v[t, :]
            for c in range(16):
                vals = plsc.load_gather(rp, [pats[c], dpat])
                jj = lax.broadcast(jnp.int32(c // 8), (16,))
                plsc.store_scatter(ob, [iidx, jj, ridx, pats[c % 8]],
                                   vals * _SCALE + peb)

    start_idx(0, 0)
    start_idx(1, 1)
    wait_idx(0, 0)
    start_gather(0, 0)

    def pair(k2, carry):
        for kk in range(2):
            k = k2 * 2 + kk

            wait_gather(k, kk)

            @pl.when(k + 2 < _NT)
            def _():
                start_idx(k + 2, kk)

            @pl.when(k + 1 < _NT)
            def _():
                wait_idx(k + 1, 1 - kk)
                start_gather(k + 1, 1 - kk)

            @pl.when(k >= 2)
            def _():
                wait_store(k - 2, kk)

            compute(k, kk, kk)
            start_store(k, kk)
        return carry

    lax.fori_loop(0, _NT // 2, pair, 0, unroll=False)
    wait_store(_NT - 2, 0)
    wait_store(_NT - 1, 1)


def kernel(token_ids, table, pe):
    tok_t = token_ids.astype(jnp.int32).T
    pe_s = pe[:_S].astype(jnp.float32)

    mesh = plsc.VectorSubcoreMesh(core_axis_name="c", subcore_axis_name="s")
    run = functools.partial(
        pl.kernel,
        mesh=mesh,
        compiler_params=pltpu.CompilerParams(use_tc_tiling_on_sc=False,
                                             needs_layout_passes=False),
        out_type=jax.ShapeDtypeStruct((_S, _DI, _JB, 8, 128), jnp.float32),
        scratch_types=[
            pltpu.VMEM((2, _W), jnp.int32),
            pltpu.VMEM((_S, _D), jnp.float32),
            pltpu.VMEM((16, 16), jnp.int32),
            pltpu.VMEM((16, 16), jnp.int32),
            pltpu.VMEM((16, 16), jnp.int32),
            pltpu.VMEM((16, 16), jnp.int32),
            pltpu.VMEM((2, _W, _D), jnp.float32),
            pltpu.VMEM((2, _DI, 2, 8, 128), jnp.float32),
            pltpu.SemaphoreType.DMA,
            pltpu.SemaphoreType.DMA,
            pltpu.SemaphoreType.DMA,
            pltpu.SemaphoreType.DMA,
            pltpu.SemaphoreType.DMA,
        ],
    )(_sc_body)
    out5 = run(tok_t, table, pe_s)
    return out5.transpose(2, 4, 0, 1, 3).reshape(_B, _S, _D)


# speedup vs baseline: 1.7285x; 1.7285x over previous
import functools
import math

import jax
import jax.numpy as jnp
from jax import lax
from jax.experimental import pallas as pl
from jax.experimental.pallas import tpu as pltpu
from jax.experimental.pallas import tpu_sc as plsc

_VOCAB = 100000
_D = 64
_B = 4096
_S = 200
_SCALE = math.sqrt(float(_D))

_NC = 2
_NS = 16
_NW = _NC * _NS
_W = 256
_NH = _B // _W
_NM = _NW // _NH
_NT = _S // _NM
_DI = _D // 8
_JB = _B // 128


def _sc_body(tok_hbm, table_hbm, pe_hbm, out_hbm, idx_v, pe_v, bidx_v,
             rot_v, div_v, rm_v, rows_v, obuf_v,
             semi, semg0, semg1, sems0, sems1):
    cid = lax.axis_index("c")
    sid = lax.axis_index("s")
    wid = sid * _NC + cid
    h = wid // _NM
    m = wid % _NM

    pltpu.sync_copy(pe_hbm, pe_v)
    iota = lax.iota(jnp.int32, 16)
    for c0 in range(16):
        bidx_v[c0, :] = iota + (c0 * 16)
    for t in range(16):
        rot = lax.rem(iota + t, jnp.full((16,), 16, jnp.int32))
        rot_v[t, :] = rot
        div_v[t, :] = lax.shift_right_logical(rot, jnp.full((16,), 3, jnp.int32))
        rm_v[t, :] = lax.rem(rot, jnp.full((16,), 8, jnp.int32))

    def s_of(k):
        return k * _NM + m

    def start_idx(k, q):
        pltpu.async_copy(tok_hbm.at[s_of(k), pl.ds(h * _W, _W)],
                         idx_v.at[q], semi)

    def wait_idx(k, q):
        pltpu.make_async_copy(tok_hbm.at[s_of(k), pl.ds(h * _W, _W)],
                              idx_v.at[q], semi).wait()

    def gsem(p):
        return (semg0, semg1)[p]

    def start_gather(k, p):
        for half in range(2):
            pltpu.async_copy(
                table_hbm.at[idx_v.at[p, pl.ds(half * 128, 128)]],
                rows_v.at[p, pl.ds(half * 128, 128)], gsem(p))

    def wait_gather(k, p):
        for half in range(2):
            pltpu.make_async_copy(
                table_hbm.at[idx_v.at[p, pl.ds(half * 128, 128)]],
                rows_v.at[p, pl.ds(half * 128, 128)], gsem(p)).wait()

    def ssem(p):
        return (sems0, sems1)[p]

    def start_store(k, p):
        pltpu.async_copy(obuf_v.at[p],
                         out_hbm.at[s_of(k), :, pl.ds(2 * h, 2)], ssem(p))

    def wait_store(k, p):
        pltpu.make_async_copy(obuf_v.at[p],
                              out_hbm.at[s_of(k), :, pl.ds(2 * h, 2)],
                              ssem(p)).wait()

    def compute(k, rows_p, obuf_p):
        rp = rows_v.at[rows_p]
        ob = obuf_v.at[obuf_p]
        s_idx = lax.broadcast(s_of(k), (16,))
        pats = tuple(bidx_v[c0, :] for c0 in range(16))

        @plsc.parallel_loop(0, 64, 1, unroll=1)
        def body_q(q):
            t = lax.rem(q, 16)
            d0i = lax.div(q, 16)
            rot = rot_v[t, :]
            dpat = rot + d0i * 16
            peb = plsc.load_gather(pe_v, [s_idx, dpat])
            iidx = div_v[t, :] + d0i * 2
            ridx = rm_v[t, :]
            for c in range(16):
                vals = plsc.load_gather(rp, [pats[c], dpat])
                jj = lax.broadcast(jnp.int32(c // 8), (16,))
                plsc.store_scatter(ob, [iidx, jj, ridx, pats[c % 8]],
                                   vals * _SCALE + peb)

    start_idx(0, 0)
    start_idx(1, 1)
    wait_idx(0, 0)
    start_gather(0, 0)

    def pair(k2, carry):
        for kk in range(2):
            k = k2 * 2 + kk

            wait_gather(k, kk)

            @pl.when(k + 2 < _NT)
            def _():
                start_idx(k + 2, kk)

            @pl.when(k + 1 < _NT)
            def _():
                wait_idx(k + 1, 1 - kk)
                start_gather(k + 1, 1 - kk)

            @pl.when(k >= 2)
            def _():
                wait_store(k - 2, kk)

            compute(k, kk, kk)
            start_store(k, kk)
        return carry

    lax.fori_loop(0, _NT // 2, pair, 0, unroll=False)
    wait_store(_NT - 2, 0)
    wait_store(_NT - 1, 1)


def kernel(token_ids, table, pe):
    tok_t = token_ids.astype(jnp.int32).T
    pe_s = pe[:_S].astype(jnp.float32)

    mesh = plsc.VectorSubcoreMesh(core_axis_name="c", subcore_axis_name="s")
    run = functools.partial(
        pl.kernel,
        mesh=mesh,
        compiler_params=pltpu.CompilerParams(use_tc_tiling_on_sc=False,
                                             needs_layout_passes=False),
        out_type=jax.ShapeDtypeStruct((_S, _DI, _JB, 8, 128), jnp.float32),
        scratch_types=[
            pltpu.VMEM((2, _W), jnp.int32),
            pltpu.VMEM((_S, _D), jnp.float32),
            pltpu.VMEM((16, 16), jnp.int32),
            pltpu.VMEM((16, 16), jnp.int32),
            pltpu.VMEM((16, 16), jnp.int32),
            pltpu.VMEM((16, 16), jnp.int32),
            pltpu.VMEM((2, _W, _D), jnp.float32),
            pltpu.VMEM((2, _DI, 2, 8, 128), jnp.float32),
            pltpu.SemaphoreType.DMA,
            pltpu.SemaphoreType.DMA,
            pltpu.SemaphoreType.DMA,
            pltpu.SemaphoreType.DMA,
            pltpu.SemaphoreType.DMA,
        ],
    )(_sc_body)
    out5 = run(tok_t, table, pe_s)
    return out5.transpose(2, 4, 0, 1, 3).reshape(_B, _S, _D)


# speedup vs baseline: 1.7906x; 1.0359x over previous
import functools
import math

import jax
import jax.numpy as jnp
from jax import lax
from jax.experimental import pallas as pl
from jax.experimental.pallas import tpu as pltpu
from jax.experimental.pallas import tpu_sc as plsc

_VOCAB = 100000
_D = 64
_B = 4096
_S = 200
_SCALE = math.sqrt(float(_D))

_NC = 2
_NS = 16
_NW = _NC * _NS
_W = 256
_NH = _B // _W
_NM = _NW // _NH
_NT = _S // _NM
_DI = _D // 8
_JB = _B // 128


def _sc_body(tok_hbm, table_hbm, pe_hbm, out_hbm, idx_v, pe_v, bidx_v,
             rot_v, div_v, rm_v, rows_v, obuf_v,
             semi, semg0, semg1, sems0, sems1):
    cid = lax.axis_index("c")
    sid = lax.axis_index("s")
    wid = sid * _NC + cid
    h = wid // _NM
    m = wid % _NM

    pltpu.sync_copy(pe_hbm, pe_v)
    iota = lax.iota(jnp.int32, 16)
    for c0 in range(16):
        bidx_v[c0, :] = iota + (c0 * 16)
    for t in range(16):
        rot = lax.rem(iota + t, jnp.full((16,), 16, jnp.int32))
        rot_v[t, :] = rot
        div_v[t, :] = lax.shift_right_logical(rot, jnp.full((16,), 3, jnp.int32))
        rm_v[t, :] = lax.rem(rot, jnp.full((16,), 8, jnp.int32))

    def s_of(k):
        return k * _NM + m

    def start_idx(k, q):
        pltpu.async_copy(tok_hbm.at[s_of(k), pl.ds(h * _W, _W)],
                         idx_v.at[q], semi)

    def wait_idx(k, q):
        pltpu.make_async_copy(tok_hbm.at[s_of(k), pl.ds(h * _W, _W)],
                              idx_v.at[q], semi).wait()

    def gsem(p):
        return (semg0, semg1)[p]

    def start_gather(k, p):
        for half in range(2):
            pltpu.async_copy(
                table_hbm.at[idx_v.at[p, pl.ds(half * 128, 128)]],
                rows_v.at[p, pl.ds(half * 128, 128)], gsem(p))

    def wait_gather(k, p):
        for half in range(2):
            pltpu.make_async_copy(
                table_hbm.at[idx_v.at[p, pl.ds(half * 128, 128)]],
                rows_v.at[p, pl.ds(half * 128, 128)], gsem(p)).wait()

    def ssem(p):
        return (sems0, sems1)[p]

    def start_store(k, p):
        pltpu.async_copy(obuf_v.at[p],
                         out_hbm.at[s_of(k), :, pl.ds(2 * h, 2)], ssem(p))

    def wait_store(k, p):
        pltpu.make_async_copy(obuf_v.at[p],
                              out_hbm.at[s_of(k), :, pl.ds(2 * h, 2)],
                              ssem(p)).wait()

    def compute(k, rows_p, obuf_p):
        rp = rows_v.at[rows_p]
        ob = obuf_v.at[obuf_p]
        s_idx = lax.broadcast(s_of(k), (16,))
        pats = tuple(bidx_v[c0, :] for c0 in range(16))

        @plsc.parallel_loop(0, 64, 1, unroll=1)
        def body_q(q):
            t = lax.rem(q, 16)
            d0i = lax.div(q, 16)
            rot = rot_v[t, :]
            dpat = rot + d0i * 16
            peb = plsc.load_gather(pe_v, [s_idx, dpat])
            iidx = div_v[t, :] + d0i * 2
            ridx = rm_v[t, :]
            for c in range(16):
                vals = plsc.load_gather(rp, [pats[c], dpat])
                jj = lax.broadcast(jnp.int32(c // 8), (16,))
                plsc.store_scatter(ob, [iidx, jj, ridx, pats[c % 8]],
                                   vals * _SCALE + peb)

    start_idx(0, 0)
    start_idx(1, 1)
    wait_idx(0, 0)
    for c0 in range(16):
        sl = pl.ds(c0 * 16, 16)
        idx_v[0, sl] = idx_v[0, sl] * 2
    start_gather(0, 0)

    def pair(k2, carry):
        for kk in range(2):
            k = k2 * 2 + kk

            wait_gather(k, kk)

            @pl.when(k + 2 < _NT)
            def _():
                start_idx(k + 2, kk)

            @pl.when(k + 1 < _NT)
            def _():
                wait_idx(k + 1, 1 - kk)
                for c0 in range(16):
                    sl = pl.ds(c0 * 16, 16)
                    idx_v[1 - kk, sl] = idx_v[1 - kk, sl] * 2
                start_gather(k + 1, 1 - kk)

            @pl.when(k >= 2)
            def _():
                wait_store(k - 2, kk)

            compute(k, kk, kk)
            start_store(k, kk)
        return carry

    lax.fori_loop(0, _NT // 2, pair, 0, unroll=False)
    wait_store(_NT - 2, 0)
    wait_store(_NT - 1, 1)


def kernel(token_ids, table, pe):
    tok_t = token_ids.astype(jnp.int32).T
    pe_s = pe[:_S].astype(jnp.float32)

    mesh = plsc.VectorSubcoreMesh(core_axis_name="c", subcore_axis_name="s")
    run = functools.partial(
        pl.kernel,
        mesh=mesh,
        compiler_params=pltpu.CompilerParams(use_tc_tiling_on_sc=False,
                                             needs_layout_passes=False),
        out_type=jax.ShapeDtypeStruct((_S, _DI, _JB, 8, 128), jnp.float32),
        scratch_types=[
            pltpu.VMEM((2, _W), jnp.int32),
            pltpu.VMEM((_S, _D), jnp.float32),
            pltpu.VMEM((16, 16), jnp.int32),
            pltpu.VMEM((16, 16), jnp.int32),
            pltpu.VMEM((16, 16), jnp.int32),
            pltpu.VMEM((16, 16), jnp.int32),
            pltpu.VMEM((2, _W, _D), jnp.float32),
            pltpu.VMEM((2, _DI, 2, 8, 128), jnp.float32),
            pltpu.SemaphoreType.DMA,
            pltpu.SemaphoreType.DMA,
            pltpu.SemaphoreType.DMA,
            pltpu.SemaphoreType.DMA,
            pltpu.SemaphoreType.DMA,
        ],
    )(_sc_body)
    table_p = jnp.pad(table, ((0, 0), (0, _D))).reshape(2 * _VOCAB, _D)
    out5 = run(tok_t, table_p, pe_s)
    return out5.transpose(2, 4, 0, 1, 3).reshape(_B, _S, _D)
